# baseline (speedup 1.0000x reference)
import functools

import jax
import jax.numpy as jnp
from jax import lax
from jax.experimental import pallas as pl
from jax.experimental.pallas import tpu as pltpu
from jax.experimental.pallas import tpu_sc as plsc

_LANES = 16
_CHUNK = 128


@functools.lru_cache(maxsize=None)
def _build_fused_kernel(B, V, D):
    info = plsc.get_sparse_core_info()
    NC, NS = info.num_cores, info.num_subcores
    DH = D // NC
    n_full = V // _CHUNK
    v_tail = V - n_full * _CHUNK
    max_p = 2 * ((n_full + 2 * NS - 1) // (2 * NS))
    b_per_s = B // NS
    n_sub = b_per_s // _CHUNK
    rpp = _CHUNK // 4
    rows_half = V // 4 + (v_tail + 3) // 4

    mesh = plsc.VectorSubcoreMesh(core_axis_name="c", subcore_axis_name="s")

    @functools.partial(
        pl.kernel,
        mesh=mesh,
        out_type=(
            jax.ShapeDtypeStruct((D, B), jnp.float32),
            jax.ShapeDtypeStruct((NC * rows_half, 2 * D), jnp.float32),
        ),
        scratch_types=[
            pltpu.VMEM((b_per_s + _LANES,), jnp.int32),
            pltpu.VMEM((b_per_s,), jnp.int32),
            pltpu.VMEM((2, DH, _CHUNK + 1), jnp.float32),
            pltpu.VMEM((2, rpp, 2 * D), jnp.float32),
            pltpu.VMEM((2, _CHUNK, 2 * D), jnp.float32),
            pltpu.VMEM((DH, b_per_s + 1), jnp.float32),
            pltpu.SemaphoreType.DMA,
            pltpu.SemaphoreType.DMA,
            pltpu.SemaphoreType.DMA,
        ],
        compiler_params=pltpu.CompilerParams(needs_layout_passes=False),
    )
    def emb(
        idx_hbm,
        table_t_hbm,
        tail_hbm,
        out_hbm,
        scr_hbm,
        idx_v,
        idx4_v,
        in_v,
        pan_v,
        rows_v,
        rows_t,
        sem_in,
        sem_out,
        sem_g,
    ):
        cid = lax.axis_index("c")
        sid = lax.axis_index("s")
        lane = lax.iota(jnp.int32, _LANES)
        d_base = pl.multiple_of(cid * DH, DH)
        row_base = cid * rows_half

        def panel_id(i):
            p = sid + i * NS
            return jnp.where(p < n_full, p, sid)

        def in_window(i, buf):
            off = pl.multiple_of(panel_id(i) * _CHUNK, _CHUNK)
            return (
                table_t_hbm.at[pl.ds(d_base, DH), pl.ds(off, _CHUNK)],
                in_v.at[buf, :, pl.ds(0, _CHUNK)],
            )

        def out_window(i, buf):
            off = pl.multiple_of(
                row_base + panel_id(i) * rpp, 8
            )
            return (pan_v.at[buf], scr_hbm.at[pl.ds(off, rpp)])

        def start_in(i, buf):
            s, d = in_window(i, buf)
            pltpu.async_copy(s, d, sem_in)

        def wait_in(i, buf):
            s, d = in_window(i, buf)
            pltpu.make_async_copy(s, d, sem_in).wait()

        def start_out(i, buf):
            s, d = out_window(i, buf)
            pltpu.async_copy(s, d, sem_out)

        def wait_out(i, buf):
            s, d = out_window(i, buf)
            pltpu.make_async_copy(s, d, sem_out).wait()

        def transpose_panel(buf, n_q):
            @plsc.parallel_loop(0, n_q)
            def tq(q):
                base_col = jnp.full((_LANES,), 4 * q, jnp.int32)
                for k in range(4):
                    col = base_col + k
                    for g in range(DH // _LANES):
                        vals = plsc.load_gather(
                            in_v.at[buf], [g * _LANES + lane, col]
                        )
                        pan_v[buf, q, pl.ds(k * DH + g * _LANES, _LANES)] = (
                            vals
                        )

        start_in(0, 0)
        start_in(1, 1)
        for i in (0, 1):
            wait_in(i, i)
            transpose_panel(i, rpp)
            start_out(i, i)
            start_in(i + 2, i)

        def body(g, _):
            for buf in (0, 1):
                i = 2 * g + buf
                wait_in(i, buf)
                wait_out(i - 2, buf)
                transpose_panel(buf, rpp)
                start_out(i, buf)
                start_in(i + 2, buf)
            return 0

        lax.fori_loop(1, max_p // 2, body, 0)
        for buf in (0, 1):
            wait_out(max_p - 2 + buf, buf)
            wait_in(max_p + buf, buf)

        if v_tail:

            @pl.when(sid == 0)
            def _():
                pltpu.sync_copy(
                    tail_hbm.at[cid],
                    scr_hbm.at[
                        pl.ds(row_base + n_full * rpp, v_tail // 4)
                    ],
                )

        plsc.subcore_barrier()

        base = sid * b_per_s
        pltpu.sync_copy(
            idx_hbm.at[pl.ds(base, b_per_s)], idx_v.at[pl.ds(0, b_per_s)]
        )

        @plsc.parallel_loop(0, b_per_s // _LANES)
        def mk_idx(g):
            sl = pl.ds(g * _LANES, _LANES)
            idx4_v[sl] = (idx_v[sl] >> 2) + row_base

        d_vecs = [d0 * _LANES + lane for d0 in range(DH // _LANES)]

        def start_gather(j):
            return pltpu.async_copy(
                scr_hbm.at[idx4_v.at[pl.ds(j * _CHUNK, _CHUNK)]],
                rows_v.at[j % 2],
                sem_g,
            )

        pending = start_gather(0)
        for j in range(n_sub):
            nxt = start_gather(j + 1) if j + 1 < n_sub else None
            pending.wait()
            buf = j % 2

            @plsc.parallel_loop(0, _CHUNK)
            def tb(b):
                gb = j * _CHUNK + b
                sub = idx_v[pl.ds(gb, _LANES)][0] & 3
                b_vec = jnp.full((_LANES,), gb, jnp.int32)
                for d0 in range(DH // _LANES):
                    vals = rows_v[
                        buf, b, pl.ds(sub * DH + d0 * _LANES, _LANES)
                    ]
                    plsc.store_scatter(rows_t, [d_vecs[d0], b_vec], vals)

            pending = nxt

        pltpu.sync_copy(
            rows_t.at[:, pl.ds(0, b_per_s)],
            out_hbm.at[pl.ds(d_base, DH), pl.ds(base, b_per_s)],
        )

    return emb


def kernel(disaster_type_idx, embedding_weight):
    (B,) = disaster_type_idx.shape
    V, D = embedding_weight.shape
    emb = _build_fused_kernel(B, V, D)
    n_full = V // 128
    v_tail = V - n_full * 128
    tail = (
        embedding_weight[n_full * 128 :, :]
        .reshape(v_tail // 4, 4, 2, D // 2)
        .transpose(2, 0, 1, 3)
        .reshape(2, v_tail // 4, 2 * D)
    )
    out_t, _ = emb(
        disaster_type_idx.astype(jnp.int32), embedding_weight.T, tail
    )
    return out_t.T
